# baseline (speedup 1.0000x reference)
import functools

import jax
import jax.numpy as jnp
from jax import lax
from jax.experimental import pallas as pl
from jax.experimental.pallas import tpu as pltpu
from jax.experimental.pallas import tpu_sc as plsc

_B = 4096
_L = 26
_EDIM = 16
_VOCAB = 5000000
_SROWS = _VOCAB // 8
_NC = 2
_NS = 16
_NW = _NC * _NS
_BPW = _B // _NW
_SUB = 16
_NCH = _BPW // _SUB
_ROWS = _L * _SUB


def _sc_body(idx_hbm, tbl_hbm, vs_hbm, c_hbm, out_hbm,
             idx_v, srow_v, rows_v, vs_v, cv_v, out_v, sem):
    wid = lax.axis_index("s") * _NC + lax.axis_index("c")

    pltpu.sync_copy(idx_hbm.at[wid], idx_v)
    pltpu.sync_copy(vs_hbm, vs_v)
    pltpu.sync_copy(c_hbm, cv_v)

    def mk_srow(i, carry):
        def one(l):
            srow_v[l, pl.ds(i * _EDIM, _EDIM)] = lax.shift_right_logical(
                idx_v[l, pl.ds(i * _EDIM, _EDIM)], 3)
        for l in range(_L):
            one(l)
        return carry
    lax.fori_loop(0, _BPW // _EDIM, mk_srow, 0)

    lanes = lax.iota(jnp.int32, _EDIM)
    cvec = cv_v[...]

    def chunk_body(c, carry):
        k0 = c * _SUB
        copies = []
        for l in range(_L):
            copies.append(pltpu.async_copy(
                tbl_hbm.at[srow_v.at[l, pl.ds(k0, _SUB)]],
                rows_v.at[pl.ds(l * _SUB, _SUB)],
                sem))
        for cp in copies:
            cp.wait()

        for g in range(_SUB // _EDIM):
            krel0 = g * _EDIM
            acc = cvec
            for l in range(_L):
                raw16 = idx_v[l, pl.ds(k0 + krel0, _EDIM)]
                col0 = (raw16 & 7) * _EDIM
                row16 = (l * _SUB + krel0) + lanes
                for d in range(_EDIM):
                    sp = vs_v[l * _EDIM + d]
                    g16 = plsc.load_gather(rows_v, [row16, col0 + d])
                    acc = acc + g16 * sp
            z = 1.0 / (1.0 + jnp.exp(-acc))
            out_v[pl.ds(k0 + krel0, _EDIM)] = z
        return carry
    lax.fori_loop(0, _NCH, chunk_body, 0)

    pltpu.sync_copy(out_v, out_hbm.at[pl.ds(wid * _BPW, _BPW)])


@jax.jit
def _run(idx, tbl, vs, cvec):
    call = functools.partial(
        pl.kernel,
        out_type=jax.ShapeDtypeStruct((_B,), jnp.float32),
        mesh=plsc.VectorSubcoreMesh(core_axis_name="c", subcore_axis_name="s"),
        compiler_params=pltpu.CompilerParams(needs_layout_passes=False),
        scratch_types=[
            pltpu.VMEM((_L, _BPW), jnp.int32),
            pltpu.VMEM((_L, _BPW), jnp.int32),
            pltpu.VMEM((_ROWS, 128), jnp.float32),
            pltpu.VMEM((_L * _EDIM, _EDIM), jnp.float32),
            pltpu.VMEM((_EDIM,), jnp.float32),
            pltpu.VMEM((_BPW,), jnp.float32),
            pltpu.SemaphoreType.DMA,
        ],
    )(_sc_body)
    return call(idx, tbl, vs, cvec)


def kernel(inputs, embed_table, W1, b1, W2, b2, W3, b3):
    W12 = jnp.dot(W1, W2)
    W3r = W3.reshape(_L, -1)
    v = jnp.dot(W3r, W12.T)
    c = jnp.sum(W3r * (jnp.dot(b1, W2) + b2)[None, :]) + b3[0]
    cvec = jnp.full((_EDIM,), c, dtype=jnp.float32)
    vs = jnp.broadcast_to(
        v.astype(jnp.float32).reshape(_L * _EDIM, 1), (_L * _EDIM, _EDIM))
    idx = inputs.astype(jnp.int32).reshape(_NW, _BPW, _L).transpose(0, 2, 1)
    tbl = embed_table.reshape(_SROWS, 128)
    out = _run(idx, tbl, vs, cvec)
    return out.reshape(_B, 1)


# speedup vs baseline: 1.0099x; 1.0099x over previous
import functools

import jax
import jax.numpy as jnp
from jax import lax
from jax.experimental import pallas as pl
from jax.experimental.pallas import tpu as pltpu
from jax.experimental.pallas import tpu_sc as plsc

_B = 4096
_L = 26
_EDIM = 16
_VOCAB = 5000000
_NC = 2
_NS = 16
_NW = _NC * _NS
_BPW = _B // _NW
_SUB = 16
_NCH = _BPW // _SUB
_ROWS = _L * _SUB


def _sc_body(idx_hbm, tbl_hbm, vs_hbm, c_hbm, out_hbm,
             idx_v, rows0_v, rows1_v, vs_v, cv_v, out_v, sem0, sem1):
    wid = lax.axis_index("s") * _NC + lax.axis_index("c")

    pltpu.sync_copy(idx_hbm.at[wid], idx_v)
    pltpu.sync_copy(vs_hbm, vs_v)
    pltpu.sync_copy(c_hbm, cv_v)

    lanes = lax.iota(jnp.int32, _EDIM)
    zeros16 = jnp.zeros((_EDIM,), jnp.int32)
    cvec = cv_v[...]
    bufs = (rows0_v, rows1_v)
    sems = (sem0, sem1)

    def fire(k0, buf, sem):
        cps = []
        for l in range(_L):
            cps.append(pltpu.async_copy(
                tbl_hbm.at[idx_v.at[l, pl.ds(k0, _SUB)]],
                buf.at[pl.ds(l * _SUB, _SUB)],
                sem))
        return cps

    def accumulate(k0, buf):
        acc = cvec
        for l in range(_L):
            row16 = (l * _SUB) + lanes
            for d in range(_EDIM):
                sp = vs_v[l * _EDIM + d]
                g16 = plsc.load_gather(buf, [row16, zeros16 + d])
                acc = acc + g16 * sp
        z = 1.0 / (1.0 + jnp.exp(-acc))
        out_v[pl.ds(k0, _SUB)] = z

    kmax = (_NCH - 1) * _SUB
    for cp in fire(0, bufs[0], sems[0]):
        cp.wait()

    def pair_body(p, carry):
        k0 = 2 * p * _SUB
        w1 = fire(jnp.minimum(k0 + _SUB, kmax), bufs[1], sems[1])
        accumulate(k0, bufs[0])
        for cp in w1:
            cp.wait()
        w0 = fire(jnp.minimum(k0 + 2 * _SUB, kmax), bufs[0], sems[0])
        accumulate(k0 + _SUB, bufs[1])
        for cp in w0:
            cp.wait()
        return carry

    lax.fori_loop(0, _NCH // 2, pair_body, 0)

    pltpu.sync_copy(out_v, out_hbm.at[pl.ds(wid * _BPW, _BPW)])


@jax.jit
def _run(idx, tbl, vs, cvec):
    call = functools.partial(
        pl.kernel,
        out_type=jax.ShapeDtypeStruct((_B,), jnp.float32),
        mesh=plsc.VectorSubcoreMesh(core_axis_name="c", subcore_axis_name="s"),
        compiler_params=pltpu.CompilerParams(
            needs_layout_passes=False,
            use_tc_tiling_on_sc=False,
        ),
        scratch_types=[
            pltpu.VMEM((_L, _BPW), jnp.int32),
            pltpu.VMEM((_ROWS, _EDIM), jnp.float32),
            pltpu.VMEM((_ROWS, _EDIM), jnp.float32),
            pltpu.VMEM((_L * _EDIM, _EDIM), jnp.float32),
            pltpu.VMEM((_EDIM,), jnp.float32),
            pltpu.VMEM((_BPW,), jnp.float32),
            pltpu.SemaphoreType.DMA,
            pltpu.SemaphoreType.DMA,
        ],
    )(_sc_body)
    return call(idx, tbl, vs, cvec)


def kernel(inputs, embed_table, W1, b1, W2, b2, W3, b3):
    W12 = jnp.dot(W1, W2)
    W3r = W3.reshape(_L, -1)
    v = jnp.dot(W3r, W12.T)
    c = jnp.sum(W3r * (jnp.dot(b1, W2) + b2)[None, :]) + b3[0]
    cvec = jnp.full((_EDIM,), c, dtype=jnp.float32)
    vs = jnp.broadcast_to(
        v.astype(jnp.float32).reshape(_L * _EDIM, 1), (_L * _EDIM, _EDIM))
    idx = inputs.astype(jnp.int32).reshape(_NW, _BPW, _L).transpose(0, 2, 1)
    out = _run(idx, embed_table, vs, cvec)
    return out.reshape(_B, 1)


# speedup vs baseline: 1.0403x; 1.0301x over previous
import functools

import jax
import jax.numpy as jnp
from jax import lax
from jax.experimental import pallas as pl
from jax.experimental.pallas import tpu as pltpu
from jax.experimental.pallas import tpu_sc as plsc

_B = 4096
_L = 26
_EDIM = 16
_VOCAB = 5000000
_SROWS = _VOCAB // 8
_NC = 2
_NS = 16
_NW = _NC * _NS
_BPW = _B // _NW
_SUB = 16
_NCH = _BPW // _SUB
_ROWS = _L * _SUB

_RCH = 128
_RW = _RCH * 8
_RPW = -(-_SROWS // _NW)
_RNCH = -(-_RPW // _RCH)


def _repack_body(p_hbm, g_hbm, buf0, buf1, buf_t, st0, st1,
                 isem0, isem1, osem0, osem1):
    wid = lax.axis_index("s") * _NC + lax.axis_index("c")
    s0 = ((wid * _SROWS) // _NW) & ~15
    smax = ((_VOCAB - _RW) // 8) & ~15
    lanes = lax.iota(jnp.int32, _EDIM)
    zeros = jnp.zeros((_EDIM,), jnp.int32)
    bufs, sts = (buf0, buf1), (st0, st1)
    isems, osems = (isem0, isem1), (osem0, osem1)

    def fire_in(c, b):
        sc = jnp.minimum(s0 + c * _RCH, smax)
        c0 = pl.multiple_of(sc * 8, 128)
        return pltpu.async_copy(
            p_hbm.at[:, pl.ds(c0, _RW)], bufs[b], isems[b]), sc

    def repack(buf, st):
        def srow(lt, carry):
            cb = lt * 8 + zeros
            for k in range(8):
                v = plsc.load_gather(buf, [lanes, cb + k])
                st[lt, pl.ds(k * _EDIM, _EDIM)] = v
            return carry
        lax.fori_loop(0, _RCH, srow, 0)

    def fire_out(st, sc, osem):
        return pltpu.async_copy(
            st, g_hbm.at[pl.ds(pl.multiple_of(sc, 16), _RCH)], osem)

    def chunk_pair(p, carry):
        in_a, sc_a = fire_in(2 * p, 0)
        in_b, sc_b = fire_in(2 * p + 1, 1)
        in_a.wait()
        repack(buf0, st0)
        out_a = fire_out(st0, sc_a, osem0)
        in_b.wait()
        repack(buf1, st1)
        out_b = fire_out(st1, sc_b, osem1)
        out_a.wait()
        out_b.wait()
        return carry
    lax.fori_loop(0, (_RNCH + 1) // 2, chunk_pair, 0)

    _TC0 = (_VOCAB // 128) * 128
    _TS0 = _TC0 // 8
    pltpu.sync_copy(p_hbm.at[:, pl.ds(_TC0, _VOCAB - _TC0)], buf_t)

    def tail_srow(lt, carry):
        cb = lt * 8 + zeros
        for k in range(8):
            v = plsc.load_gather(buf_t, [lanes, cb + k])
            st0[lt, pl.ds(k * _EDIM, _EDIM)] = v
        return carry
    lax.fori_loop(0, _SROWS - _TS0, tail_srow, 0)
    pltpu.sync_copy(st0.at[pl.ds(0, _SROWS - _TS0)],
                    g_hbm.at[pl.ds(_TS0, _SROWS - _TS0)])


def _sc_body(idx_hbm, tbl_hbm, vs_hbm, c_hbm, out_hbm,
             idx_v, srow_v, rows_v, vs_v, cv_v, out_v, sem):
    wid = lax.axis_index("s") * _NC + lax.axis_index("c")

    pltpu.sync_copy(idx_hbm.at[wid], idx_v)
    pltpu.sync_copy(vs_hbm, vs_v)
    pltpu.sync_copy(c_hbm, cv_v)

    def mk_srow(i, carry):
        def one(l):
            srow_v[l, pl.ds(i * _EDIM, _EDIM)] = lax.shift_right_logical(
                idx_v[l, pl.ds(i * _EDIM, _EDIM)], 3)
        for l in range(_L):
            one(l)
        return carry
    lax.fori_loop(0, _BPW // _EDIM, mk_srow, 0)

    lanes = lax.iota(jnp.int32, _EDIM)
    cvec = cv_v[...]

    def chunk_body(c, carry):
        k0 = c * _SUB
        copies = []
        for l in range(_L):
            copies.append(pltpu.async_copy(
                tbl_hbm.at[srow_v.at[l, pl.ds(k0, _SUB)]],
                rows_v.at[pl.ds(l * _SUB, _SUB)],
                sem))
        for cp in copies:
            cp.wait()

        acc = cvec
        for l in range(_L):
            raw16 = idx_v[l, pl.ds(k0, _EDIM)]
            col0 = (raw16 & 7) * _EDIM
            row16 = (l * _SUB) + lanes
            for d in range(_EDIM):
                sp = vs_v[l * _EDIM + d]
                g16 = plsc.load_gather(rows_v, [row16, col0 + d])
                acc = acc + g16 * sp
        z = 1.0 / (1.0 + jnp.exp(-acc))
        out_v[pl.ds(k0, _EDIM)] = z
        return carry
    lax.fori_loop(0, _NCH, chunk_body, 0)

    pltpu.sync_copy(out_v, out_hbm.at[pl.ds(wid * _BPW, _BPW)])


@jax.jit
def _run(idx, tblT, vs, cvec):
    repack = functools.partial(
        pl.kernel,
        out_type=jax.ShapeDtypeStruct((_SROWS, 128), jnp.float32),
        mesh=plsc.VectorSubcoreMesh(core_axis_name="c", subcore_axis_name="s"),
        compiler_params=pltpu.CompilerParams(
            needs_layout_passes=False,
            use_tc_tiling_on_sc=True,
        ),
        scratch_types=[
            pltpu.VMEM((_EDIM, _RW), jnp.float32),
            pltpu.VMEM((_EDIM, _RW), jnp.float32),
            pltpu.VMEM((_EDIM, _VOCAB - (_VOCAB // 128) * 128), jnp.float32),
            pltpu.VMEM((_RCH, 128), jnp.float32),
            pltpu.VMEM((_RCH, 128), jnp.float32),
            pltpu.SemaphoreType.DMA,
            pltpu.SemaphoreType.DMA,
            pltpu.SemaphoreType.DMA,
            pltpu.SemaphoreType.DMA,
        ],
    )(_repack_body)
    g = repack(tblT)
    call = functools.partial(
        pl.kernel,
        out_type=jax.ShapeDtypeStruct((_B,), jnp.float32),
        mesh=plsc.VectorSubcoreMesh(core_axis_name="c", subcore_axis_name="s"),
        compiler_params=pltpu.CompilerParams(
            needs_layout_passes=False,
            use_tc_tiling_on_sc=True,
        ),
        scratch_types=[
            pltpu.VMEM((_L, _BPW), jnp.int32),
            pltpu.VMEM((_L, _BPW), jnp.int32),
            pltpu.VMEM((_ROWS, 128), jnp.float32),
            pltpu.VMEM((_L * _EDIM, _EDIM), jnp.float32),
            pltpu.VMEM((_EDIM,), jnp.float32),
            pltpu.VMEM((_BPW,), jnp.float32),
            pltpu.SemaphoreType.DMA,
        ],
    )(_sc_body)
    return call(idx, g, vs, cvec)


def kernel(inputs, embed_table, W1, b1, W2, b2, W3, b3):
    W12 = jnp.dot(W1, W2)
    W3r = W3.reshape(_L, -1)
    v = jnp.dot(W3r, W12.T)
    c = jnp.sum(W3r * (jnp.dot(b1, W2) + b2)[None, :]) + b3[0]
    cvec = jnp.full((_EDIM,), c, dtype=jnp.float32)
    vs = jnp.broadcast_to(
        v.astype(jnp.float32).reshape(_L * _EDIM, 1), (_L * _EDIM, _EDIM))
    idx = inputs.astype(jnp.int32).reshape(_NW, _BPW, _L).transpose(0, 2, 1)
    tblT = embed_table.T
    out = _run(idx, tblT, vs, cvec)
    return out.reshape(_B, 1)


# speedup vs baseline: 1.0952x; 1.0528x over previous
import functools

import jax
import jax.numpy as jnp
from jax import lax
from jax.experimental import pallas as pl
from jax.experimental.pallas import tpu as pltpu
from jax.experimental.pallas import tpu_sc as plsc

_B = 4096
_L = 26
_EDIM = 16
_VOCAB = 5000000
_SROWS = _VOCAB // 8
_NC = 2
_NS = 16
_NW = _NC * _NS
_BPW = _B // _NW
_SUB = 16
_NCH = _BPW // _SUB
_ROWS = _L * _SUB

_TW = 2048
_TG = -(-_VOCAB // _TW)
_GR = _TW // 8


def _tc_repack_body(p_ref, g_ref):
    xt = p_ref[...].T.reshape(_GR, 8, _EDIM)
    for k in range(8):
        g_ref[:, pl.ds(k * _EDIM, _EDIM)] = xt[:, k, :]


def _sc_body(idx_hbm, tbl_hbm, vs_hbm, c_hbm, out_hbm,
             idx_v, srow_v, rows_v, vs_v, cv_v, out_v, sem):
    wid = lax.axis_index("s") * _NC + lax.axis_index("c")

    pltpu.sync_copy(idx_hbm.at[wid], idx_v)
    pltpu.sync_copy(vs_hbm, vs_v)
    pltpu.sync_copy(c_hbm, cv_v)

    def mk_srow(i, carry):
        def one(l):
            srow_v[l, pl.ds(i * _EDIM, _EDIM)] = lax.shift_right_logical(
                idx_v[l, pl.ds(i * _EDIM, _EDIM)], 3)
        for l in range(_L):
            one(l)
        return carry
    lax.fori_loop(0, _BPW // _EDIM, mk_srow, 0)

    lanes = lax.iota(jnp.int32, _EDIM)
    cvec = cv_v[...]

    def chunk_body(c, carry):
        k0 = c * _SUB
        copies = []
        for l in range(_L):
            copies.append(pltpu.async_copy(
                tbl_hbm.at[srow_v.at[l, pl.ds(k0, _SUB)]],
                rows_v.at[pl.ds(l * _SUB, _SUB)],
                sem))
        for cp in copies:
            cp.wait()

        acc = cvec
        for l in range(_L):
            raw16 = idx_v[l, pl.ds(k0, _EDIM)]
            col0 = (raw16 & 7) * _EDIM
            row16 = (l * _SUB) + lanes
            for d in range(_EDIM):
                sp = vs_v[l * _EDIM + d]
                g16 = plsc.load_gather(rows_v, [row16, col0 + d])
                acc = acc + g16 * sp
        z = 1.0 / (1.0 + jnp.exp(-acc))
        out_v[pl.ds(k0, _EDIM)] = z
        return carry
    lax.fori_loop(0, _NCH, chunk_body, 0)

    pltpu.sync_copy(out_v, out_hbm.at[pl.ds(wid * _BPW, _BPW)])


@jax.jit
def _run(idx, tblT, vs, cvec):
    g = pl.pallas_call(
        _tc_repack_body,
        grid=(_TG,),
        in_specs=[pl.BlockSpec((_EDIM, _TW), lambda i: (0, i))],
        out_specs=pl.BlockSpec((_GR, 128), lambda i: (i, 0)),
        out_shape=jax.ShapeDtypeStruct((_SROWS, 128), jnp.float32),
    )(tblT)
    call = functools.partial(
        pl.kernel,
        out_type=jax.ShapeDtypeStruct((_B,), jnp.float32),
        mesh=plsc.VectorSubcoreMesh(core_axis_name="c", subcore_axis_name="s"),
        compiler_params=pltpu.CompilerParams(
            needs_layout_passes=False,
            use_tc_tiling_on_sc=True,
        ),
        scratch_types=[
            pltpu.VMEM((_L, _BPW), jnp.int32),
            pltpu.VMEM((_L, _BPW), jnp.int32),
            pltpu.VMEM((_ROWS, 128), jnp.float32),
            pltpu.VMEM((_L * _EDIM, _EDIM), jnp.float32),
            pltpu.VMEM((_EDIM,), jnp.float32),
            pltpu.VMEM((_BPW,), jnp.float32),
            pltpu.SemaphoreType.DMA,
        ],
    )(_sc_body)
    return call(idx, g, vs, cvec)


def kernel(inputs, embed_table, W1, b1, W2, b2, W3, b3):
    W12 = jnp.dot(W1, W2)
    W3r = W3.reshape(_L, -1)
    v = jnp.dot(W3r, W12.T)
    c = jnp.sum(W3r * (jnp.dot(b1, W2) + b2)[None, :]) + b3[0]
    cvec = jnp.full((_EDIM,), c, dtype=jnp.float32)
    vs = jnp.broadcast_to(
        v.astype(jnp.float32).reshape(_L * _EDIM, 1), (_L * _EDIM, _EDIM))
    idx = inputs.astype(jnp.int32).reshape(_NW, _BPW, _L).transpose(0, 2, 1)
    tblT = embed_table.T
    out = _run(idx, tblT, vs, cvec)
    return out.reshape(_B, 1)


# speedup vs baseline: 1.6277x; 1.4863x over previous
import functools

import jax
import jax.numpy as jnp
from jax import lax
from jax.experimental import pallas as pl
from jax.experimental.pallas import tpu as pltpu
from jax.experimental.pallas import tpu_sc as plsc

_B = 4096
_L = 26
_EDIM = 16
_VOCAB = 5000000
_SROWS = _VOCAB // 8
_NC = 2
_NS = 16
_NW = _NC * _NS
_BPW = _B // _NW
_SUB = 16
_NCH = _BPW // _SUB
_ROWS = _L * _SUB

_TW = 8192
_TG = -(-_VOCAB // _TW)
_GR = _TW // 8


def _tc_repack_body(p_ref, g_ref):
    xt = p_ref[...].T.reshape(_GR, 8, _EDIM)
    for k in range(8):
        g_ref[:, pl.ds(k * _EDIM, _EDIM)] = xt[:, k, :]


def _sc_body(idx_hbm, tbl_hbm, vs_hbm, c_hbm, out_hbm,
             idx_v, srow_v, rows_v, vs_v, cv_v, out_v, sem):
    wid = lax.axis_index("s") * _NC + lax.axis_index("c")

    pltpu.sync_copy(idx_hbm.at[wid], idx_v)
    pltpu.sync_copy(vs_hbm, vs_v)
    pltpu.sync_copy(c_hbm, cv_v)

    def mk_srow(i, carry):
        def one(l):
            srow_v[l, pl.ds(i * _EDIM, _EDIM)] = lax.shift_right_logical(
                idx_v[l, pl.ds(i * _EDIM, _EDIM)], 3)
        for l in range(_L):
            one(l)
        return carry
    lax.fori_loop(0, _BPW // _EDIM, mk_srow, 0)

    lanes = lax.iota(jnp.int32, _EDIM)
    cvec = cv_v[...]

    def chunk_body(c, carry):
        k0 = c * _SUB
        copies = []
        for l in range(_L):
            copies.append(pltpu.async_copy(
                tbl_hbm.at[srow_v.at[l, pl.ds(k0, _SUB)]],
                rows_v.at[pl.ds(l * _SUB, _SUB)],
                sem))
        for cp in copies:
            cp.wait()

        acc = cvec
        for l in range(_L):
            raw16 = idx_v[l, pl.ds(k0, _EDIM)]
            col0 = (raw16 & 7) * _EDIM
            row16 = (l * _SUB) + lanes
            for d in range(_EDIM):
                sp = vs_v[l * _EDIM + d]
                g16 = plsc.load_gather(rows_v, [row16, col0 + d])
                acc = acc + g16 * sp
        z = 1.0 / (1.0 + jnp.exp(-acc))
        out_v[pl.ds(k0, _EDIM)] = z
        return carry
    lax.fori_loop(0, _NCH, chunk_body, 0)

    pltpu.sync_copy(out_v, out_hbm.at[pl.ds(wid * _BPW, _BPW)])


@jax.jit
def _run(idx, tblT, vs, cvec):
    g = pl.pallas_call(
        _tc_repack_body,
        grid=(_TG,),
        in_specs=[pl.BlockSpec((_EDIM, _TW), lambda i: (0, i))],
        out_specs=pl.BlockSpec((_GR, 128), lambda i: (i, 0)),
        out_shape=jax.ShapeDtypeStruct((_SROWS, 128), jnp.float32),
        compiler_params=pltpu.CompilerParams(
            dimension_semantics=("parallel",)),
    )(tblT)
    call = functools.partial(
        pl.kernel,
        out_type=jax.ShapeDtypeStruct((_B,), jnp.float32),
        mesh=plsc.VectorSubcoreMesh(core_axis_name="c", subcore_axis_name="s"),
        compiler_params=pltpu.CompilerParams(
            needs_layout_passes=False,
            use_tc_tiling_on_sc=True,
        ),
        scratch_types=[
            pltpu.VMEM((_L, _BPW), jnp.int32),
            pltpu.VMEM((_L, _BPW), jnp.int32),
            pltpu.VMEM((_ROWS, 128), jnp.float32),
            pltpu.VMEM((_L * _EDIM, _EDIM), jnp.float32),
            pltpu.VMEM((_EDIM,), jnp.float32),
            pltpu.VMEM((_BPW,), jnp.float32),
            pltpu.SemaphoreType.DMA,
        ],
    )(_sc_body)
    return call(idx, g, vs, cvec)


def kernel(inputs, embed_table, W1, b1, W2, b2, W3, b3):
    W12 = jnp.dot(W1, W2)
    W3r = W3.reshape(_L, -1)
    v = jnp.dot(W3r, W12.T)
    c = jnp.sum(W3r * (jnp.dot(b1, W2) + b2)[None, :]) + b3[0]
    cvec = jnp.full((_EDIM,), c, dtype=jnp.float32)
    vs = jnp.broadcast_to(
        v.astype(jnp.float32).reshape(_L * _EDIM, 1), (_L * _EDIM, _EDIM))
    idx = inputs.astype(jnp.int32).reshape(_NW, _BPW, _L).transpose(0, 2, 1)
    tblT = embed_table.T
    out = _run(idx, tblT, vs, cvec)
    return out.reshape(_B, 1)


# speedup vs baseline: 1.6895x; 1.0380x over previous
import functools

import jax
import jax.numpy as jnp
from jax import lax
from jax.experimental import pallas as pl
from jax.experimental.pallas import tpu as pltpu
from jax.experimental.pallas import tpu_sc as plsc

_B = 4096
_L = 26
_EDIM = 16
_VOCAB = 5000000
_SROWS = _VOCAB // 8
_NC = 2
_NS = 16
_NW = _NC * _NS
_BPW = _B // _NW
_SUB = 16
_NCH = _BPW // _SUB
_ROWS = _L * _SUB

_TW = 32768
_TG = -(-_VOCAB // _TW)
_GR = _TW // 8


def _tc_repack_body(p_ref, g_ref):
    xt = p_ref[...].T.reshape(_GR, 8, _EDIM)
    for k in range(8):
        g_ref[:, pl.ds(k * _EDIM, _EDIM)] = xt[:, k, :]


def _sc_body(idx_hbm, tbl_hbm, vs_hbm, c_hbm, out_hbm,
             idx_v, srow_v, rows_v, vs_v, cv_v, out_v, sem):
    wid = lax.axis_index("s") * _NC + lax.axis_index("c")

    pltpu.sync_copy(idx_hbm.at[wid], idx_v)
    pltpu.sync_copy(vs_hbm, vs_v)
    pltpu.sync_copy(c_hbm, cv_v)

    def mk_srow(i, carry):
        def one(l):
            srow_v[l, pl.ds(i * _EDIM, _EDIM)] = lax.shift_right_logical(
                idx_v[l, pl.ds(i * _EDIM, _EDIM)], 3)
        for l in range(_L):
            one(l)
        return carry
    lax.fori_loop(0, _BPW // _EDIM, mk_srow, 0)

    lanes = lax.iota(jnp.int32, _EDIM)
    cvec = cv_v[...]

    def chunk_body(c, carry):
        k0 = c * _SUB
        copies = []
        for l in range(_L):
            copies.append(pltpu.async_copy(
                tbl_hbm.at[srow_v.at[l, pl.ds(k0, _SUB)]],
                rows_v.at[pl.ds(l * _SUB, _SUB)],
                sem))
        for cp in copies:
            cp.wait()

        acc = cvec
        for l in range(_L):
            raw16 = idx_v[l, pl.ds(k0, _EDIM)]
            col0 = (raw16 & 7) * _EDIM
            row16 = (l * _SUB) + lanes
            for d in range(_EDIM):
                sp = vs_v[l * _EDIM + d]
                g16 = plsc.load_gather(rows_v, [row16, col0 + d])
                acc = acc + g16 * sp
        z = 1.0 / (1.0 + jnp.exp(-acc))
        out_v[pl.ds(k0, _EDIM)] = z
        return carry
    lax.fori_loop(0, _NCH, chunk_body, 0)

    pltpu.sync_copy(out_v, out_hbm.at[pl.ds(wid * _BPW, _BPW)])


@jax.jit
def _run(idx, tblT, vs, cvec):
    g = pl.pallas_call(
        _tc_repack_body,
        grid=(_TG,),
        in_specs=[pl.BlockSpec((_EDIM, _TW), lambda i: (0, i))],
        out_specs=pl.BlockSpec((_GR, 128), lambda i: (i, 0)),
        out_shape=jax.ShapeDtypeStruct((_SROWS, 128), jnp.float32),
        compiler_params=pltpu.CompilerParams(
            dimension_semantics=("parallel",)),
    )(tblT)
    call = functools.partial(
        pl.kernel,
        out_type=jax.ShapeDtypeStruct((_B,), jnp.float32),
        mesh=plsc.VectorSubcoreMesh(core_axis_name="c", subcore_axis_name="s"),
        compiler_params=pltpu.CompilerParams(
            needs_layout_passes=False,
            use_tc_tiling_on_sc=True,
        ),
        scratch_types=[
            pltpu.VMEM((_L, _BPW), jnp.int32),
            pltpu.VMEM((_L, _BPW), jnp.int32),
            pltpu.VMEM((_ROWS, 128), jnp.float32),
            pltpu.VMEM((_L * _EDIM, _EDIM), jnp.float32),
            pltpu.VMEM((_EDIM,), jnp.float32),
            pltpu.VMEM((_BPW,), jnp.float32),
            pltpu.SemaphoreType.DMA,
        ],
    )(_sc_body)
    return call(idx, g, vs, cvec)


def kernel(inputs, embed_table, W1, b1, W2, b2, W3, b3):
    W12 = jnp.dot(W1, W2)
    W3r = W3.reshape(_L, -1)
    v = jnp.dot(W3r, W12.T)
    c = jnp.sum(W3r * (jnp.dot(b1, W2) + b2)[None, :]) + b3[0]
    cvec = jnp.full((_EDIM,), c, dtype=jnp.float32)
    vs = jnp.broadcast_to(
        v.astype(jnp.float32).reshape(_L * _EDIM, 1), (_L * _EDIM, _EDIM))
    idx = inputs.astype(jnp.int32).reshape(_NW, _BPW, _L).transpose(0, 2, 1)
    tblT = embed_table.T
    out = _run(idx, tblT, vs, cvec)
    return out.reshape(_B, 1)
